# baseline (speedup 1.0000x reference)
import functools

import jax
import jax.numpy as jnp
from jax import lax
from jax.experimental import pallas as pl
from jax.experimental.pallas import tpu as pltpu
from jax.experimental.pallas import tpu_sc as plsc

_N = 100000
_E = 3200000
_G = 1000
_D = 16

_NC = 2
_NS = 16
_NW = _NC * _NS

_CH = 128
_T1 = 4
_T2 = 6
_CPS = 792
_EPAD = _NW * _CPS * _CH
_NCHUNK = _EPAD // _CH
_NCHUNKP = _NCHUNK + 8

_R = 100096
_RPS = _R // _NS
_ZC = _RPS // 4
_ZW = 3136

_BN = 4000

_RP = 102400
_NPCH = _RP // _CH
_PPS = _NPCH // _NS
_GP = 1024
_GPS = _GP // _NS


def _mesh():
    return plsc.VectorSubcoreMesh(core_axis_name="c", subcore_axis_name="s")


def _make_edge_pass(with_cnt: bool, t: int):
    outer = _CPS // t
    assert outer % 2 == 0
    out_type = [jax.ShapeDtypeStruct((_NC, _R, _D), jnp.float32)]
    if with_cnt:
        out_type.append(jax.ShapeDtypeStruct((_NC, _R), jnp.float32))
    scratch = []
    for _ in range(2):
        scratch += [
            pltpu.VMEM((t, _CH), jnp.int32),
            pltpu.VMEM((t, _CH), jnp.int32),
            pltpu.VMEM((t * _CH, _D), jnp.float32),
        ]
    scratch.append(pltpu.VMEM_SHARED((_R, _D), jnp.float32))
    if with_cnt:
        scratch += [
            pltpu.VMEM((_CH,), jnp.float32),
            pltpu.VMEM((_ZW,), jnp.float32),
            pltpu.VMEM_SHARED((_R + 16,), jnp.float32),
        ]
    scratch += [pltpu.SemaphoreType.DMA] * (8 if with_cnt else 6)

    @functools.partial(
        pl.kernel, mesh=_mesh(), out_type=out_type, scratch_types=scratch,
        compiler_params=pltpu.CompilerParams(use_tc_tiling_on_sc=False))
    def edge_pass(table, srcg, dstg, *rest):
        rest = list(rest)
        acc_out = rest.pop(0)
        cnt_out = rest.pop(0) if with_cnt else None
        idx_s = [rest[0], rest[3]]
        idx_d = [rest[1], rest[4]]
        rows = [rest[2], rest[5]]
        acc_sh = rest[6]
        if with_cnt:
            ones, zbuf, cnt_sh = rest[7], rest[8], rest[9]
            isem = rest[10:12]
            gsem = rest[12:14]
            ssem = rest[14:16]
            csem = rest[16:18]
        else:
            isem = rest[7:9]
            gsem = rest[9:11]
            ssem = rest[11:13]
        c = lax.axis_index("c")
        s = lax.axis_index("s")
        w = c * _NS + s

        @pl.loop(0, _ZC)
        def _(i):
            rows[0][i, :] = jnp.zeros((_D,), jnp.float32)

        for k in range(4):
            pltpu.sync_copy(rows[0].at[pl.ds(0, _ZC)],
                            acc_sh.at[pl.ds(s * _RPS + k * _ZC, _ZC)])
        if with_cnt:
            @pl.loop(0, _ZW // 16)
            def _(i):
                zbuf[pl.ds(i * 16, 16)] = jnp.zeros((16,), jnp.float32)

            for k in range(2):
                pltpu.sync_copy(zbuf,
                                cnt_sh.at[pl.ds(s * _RPS + k * _ZW, _ZW)])

            @pl.loop(0, _CH // 16)
            def _(i):
                ones[pl.ds(i * 16, 16)] = jnp.ones((16,), jnp.float32)

        plsc.subcore_barrier()

        base = w * _CPS

        def load_idx(b, it):
            c0 = base + it * t
            pltpu.async_copy(srcg.at[pl.ds(c0, t)], idx_s[b], isem[b])
            pltpu.async_copy(dstg.at[pl.ds(c0, t)], idx_d[b], isem[b])

        def wait_idx(b):
            pltpu.make_async_copy(srcg.at[pl.ds(0, t)], idx_s[b],
                                  isem[b]).wait()
            pltpu.make_async_copy(dstg.at[pl.ds(0, t)], idx_d[b],
                                  isem[b]).wait()

        def run_gathers(b):
            hs = [pltpu.async_copy(table.at[idx_s[b].at[j]],
                                   rows[b].at[pl.ds(j * _CH, _CH)], gsem[b])
                  for j in range(t)]
            for h in hs:
                h.wait()

        def fire_scatters(b):
            for j in range(t):
                pltpu.async_copy(rows[b].at[pl.ds(j * _CH, _CH)],
                                 acc_sh.at[idx_d[b].at[j]], ssem[b],
                                 add=True)
                if with_cnt:
                    pltpu.async_copy(ones, cnt_sh.at[idx_d[b].at[j]],
                                     csem[b], add=True)

        def drain_scatters(b):
            for j in range(t):
                pltpu.make_async_copy(rows[b].at[pl.ds(j * _CH, _CH)],
                                      acc_sh.at[idx_d[b].at[j]],
                                      ssem[b]).wait()
                if with_cnt:
                    pltpu.make_async_copy(ones, cnt_sh.at[idx_d[b].at[j]],
                                          csem[b]).wait()

        load_idx(0, 0)

        @pl.loop(0, outer // 2)
        def _(p):
            it0 = p * 2
            wait_idx(0)
            run_gathers(0)

            @pl.when(p > 0)
            def _():
                drain_scatters(1)

            load_idx(1, it0 + 1)
            fire_scatters(0)
            wait_idx(1)
            run_gathers(1)
            drain_scatters(0)
            load_idx(0, it0 + 2)
            fire_scatters(1)

        drain_scatters(1)
        wait_idx(0)
        plsc.subcore_barrier()

        r0 = s * _RPS
        pltpu.sync_copy(acc_sh.at[pl.ds(r0, _RPS)],
                        acc_out.at[c, pl.ds(r0, _RPS)])
        if with_cnt:
            pltpu.sync_copy(cnt_sh.at[pl.ds(r0, _RPS)],
                            cnt_out.at[c, pl.ds(r0, _RPS)])

    return edge_pass


_edge_pass_cnt = _make_edge_pass(True, _T1)
_edge_pass = _make_edge_pass(False, _T2)


@functools.partial(pl.kernel, mesh=_mesh(),
                   out_type=jax.ShapeDtypeStruct((_GP,), jnp.float32),
                   compiler_params=pltpu.CompilerParams(
                       use_tc_tiling_on_sc=False),
                   scratch_types=[
                       pltpu.VMEM((1, _CH), jnp.float32),
                       pltpu.VMEM((1, _CH), jnp.int32),
                       pltpu.VMEM((_CH,), jnp.float32),
                       pltpu.VMEM((_GPS,), jnp.float32),
                       pltpu.VMEM((_GPS,), jnp.float32),
                       pltpu.VMEM((16,), jnp.float32),
                       pltpu.VMEM((_GPS,), jnp.float32),
                       pltpu.VMEM_SHARED((_GP,), jnp.float32),
                       pltpu.VMEM_SHARED((_GP,), jnp.float32),
                   ])
def _pool_pass(tg, bg, blin_h, out, tbuf, ibuf, ones, pbuf, cbuf, bbuf, obuf,
               psum_sh, pcnt_sh):
    c = lax.axis_index("c")
    s = lax.axis_index("s")

    @pl.when(c == 0)
    def _():
        @pl.loop(0, _CH // 16)
        def _(i):
            ones[pl.ds(i * 16, 16)] = jnp.ones((16,), jnp.float32)

        @pl.loop(0, _GPS // 16)
        def _(i):
            pbuf[pl.ds(i * 16, 16)] = jnp.zeros((16,), jnp.float32)

        pltpu.sync_copy(pbuf, psum_sh.at[pl.ds(s * _GPS, _GPS)])
        pltpu.sync_copy(pbuf, pcnt_sh.at[pl.ds(s * _GPS, _GPS)])
        plsc.subcore_barrier()

        @pl.loop(0, _PPS)
        def _(it):
            ch = s * _PPS + it
            pltpu.sync_copy(tg.at[ch], tbuf)
            pltpu.sync_copy(bg.at[ch], ibuf)
            pltpu.sync_copy(tbuf.at[0], psum_sh.at[ibuf.at[0]], add=True)
            pltpu.sync_copy(ones, pcnt_sh.at[ibuf.at[0]], add=True)

        plsc.subcore_barrier()

        pltpu.sync_copy(psum_sh.at[pl.ds(s * _GPS, _GPS)], pbuf)
        pltpu.sync_copy(pcnt_sh.at[pl.ds(s * _GPS, _GPS)], cbuf)
        pltpu.sync_copy(blin_h, bbuf)
        b = bbuf[pl.ds(0, 16)][0]
        for k in range(_GPS // 16):
            obuf[pl.ds(k * 16, 16)] = (
                pbuf[pl.ds(k * 16, 16)]
                / jnp.maximum(cbuf[pl.ds(k * 16, 16)], 1.0) + b)
        pltpu.sync_copy(obuf, out.at[pl.ds(s * _GPS, _GPS)])


def _p1_body(x_ref, w_ref, o_ref):
    o_ref[:] = jnp.dot(x_ref[:], w_ref[:], preferred_element_type=jnp.float32)


def _qp_body(acc_ref, c0_ref, c1_ref, x_ref, w1rt_ref, b1_ref, w2lt_ref,
             w2rt_ref, b2_ref, a2_ref, r2_ref):
    cnt = jnp.maximum(c0_ref[:] + c1_ref[:], 1.0)
    mean = (acc_ref[0] + acc_ref[1]) / cnt
    h1 = mean + jnp.dot(x_ref[:], w1rt_ref[:],
                        preferred_element_type=jnp.float32) + b1_ref[:]
    h1 = jnp.maximum(h1, 0.0)
    a2_ref[:] = jnp.dot(h1, w2lt_ref[:], preferred_element_type=jnp.float32)
    r2_ref[:] = jnp.dot(h1, w2rt_ref[:],
                        preferred_element_type=jnp.float32) + b2_ref[:]


def _q2_body(acc_ref, c0_ref, c1_ref, r2_ref, wlin_ref, t_ref):
    cnt = jnp.maximum(c0_ref[:] + c1_ref[:], 1.0)
    h2 = (acc_ref[0] + acc_ref[1]) / cnt + r2_ref[:]
    t_ref[:] = jnp.sum(h2 * wlin_ref[:], axis=1, keepdims=True)


def _row_spec():
    return pl.BlockSpec((_BN, _D), lambda i: (i, 0))


def _vec_spec():
    return pl.BlockSpec((_BN, 1), lambda i: (i, 0))


def _w_spec():
    return pl.BlockSpec((_D, _D), lambda i: (0, 0))


def _b_spec():
    return pl.BlockSpec((1, _D), lambda i: (0, 0))


def _acc_spec():
    return pl.BlockSpec((_NC, _BN, _D), lambda i: (0, i, 0))


def kernel(matrix_encodings, edge_index, batch, W1l, b1l, W1r, W2l, b2l, W2r,
           Wlin, blin):
    x = matrix_encodings
    f32, i32 = jnp.float32, jnp.int32
    grid = (_N // _BN,)

    npad = _NCHUNKP * _CH - _E
    srcg = jnp.concatenate(
        [edge_index[0], jnp.zeros((npad,), i32)]).reshape(_NCHUNKP, _CH)
    dstg = jnp.concatenate(
        [edge_index[1], _N + (jnp.arange(npad, dtype=i32) % (_R - _N))]
    ).reshape(_NCHUNKP, _CH)

    a1 = pl.pallas_call(
        _p1_body, grid=grid,
        in_specs=[_row_spec(), _w_spec()],
        out_specs=_row_spec(),
        out_shape=jax.ShapeDtypeStruct((_N, _D), f32),
    )(x, W1l.T)

    acc1, cnt = _edge_pass_cnt(a1, srcg, dstg)

    a2, r2 = pl.pallas_call(
        _qp_body, grid=grid,
        in_specs=[_acc_spec(), _vec_spec(), _vec_spec(), _row_spec(),
                  _w_spec(), _b_spec(), _w_spec(), _w_spec(), _b_spec()],
        out_specs=[_row_spec(), _row_spec()],
        out_shape=[jax.ShapeDtypeStruct((_N, _D), f32),
                   jax.ShapeDtypeStruct((_N, _D), f32)],
    )(acc1, cnt[0].reshape(_R, 1), cnt[1].reshape(_R, 1), x, W1r.T,
      b1l.reshape(1, _D), W2l.T, W2r.T, b2l.reshape(1, _D))

    (acc2,) = _edge_pass(a2, srcg, dstg)

    t = pl.pallas_call(
        _q2_body, grid=grid,
        in_specs=[_acc_spec(), _vec_spec(), _vec_spec(), _row_spec(),
                  _b_spec()],
        out_specs=_vec_spec(),
        out_shape=jax.ShapeDtypeStruct((_N, 1), f32),
    )(acc2, cnt[0].reshape(_R, 1), cnt[1].reshape(_R, 1), r2, Wlin)

    tg = jnp.concatenate(
        [t.reshape(_N), jnp.zeros((_RP - _N,), f32)]).reshape(_NPCH, 1, _CH)
    bg = jnp.concatenate(
        [batch, jnp.full((_RP - _N,), _G, i32)]).reshape(_NPCH, 1, _CH)
    pooled = _pool_pass(tg, bg, jnp.pad(blin, (0, 15)))
    return pooled[:_G].reshape(_G, 1)


# speedup vs baseline: 1.1404x; 1.1404x over previous
import functools

import jax
import jax.numpy as jnp
from jax import lax
from jax.experimental import pallas as pl
from jax.experimental.pallas import tpu as pltpu
from jax.experimental.pallas import tpu_sc as plsc

_N = 100000
_E = 3200000
_G = 1000
_D = 16

_NC = 2
_NS = 16
_NW = _NC * _NS

_CH = 128
_T1 = 4
_T2 = 6
_CPS = 792
_EPAD = _NW * _CPS * _CH
_NCHUNK = _EPAD // _CH
_NCHUNKP = _NCHUNK + 8

_R = 100096
_RPS = _R // _NS
_ZC = _RPS // 4
_ZW = 3136

_BN = 4000

_RP = 102400
_NPCH = _RP // _CH
_PPS = _NPCH // _NS
_GP = 1024
_GPS = _GP // _NS


def _mesh():
    return plsc.VectorSubcoreMesh(core_axis_name="c", subcore_axis_name="s")


def _make_edge_pass(with_cnt: bool, t: int, cps0: int, cps1: int):
    assert cps0 + cps1 == 2 * _CPS
    assert cps0 % (2 * t) == 0 and cps1 % (2 * t) == 0
    out_type = [jax.ShapeDtypeStruct((_NC, _R, _D), jnp.float32)]
    if with_cnt:
        out_type.append(jax.ShapeDtypeStruct((_NC, _R), jnp.float32))
    scratch = []
    for _ in range(2):
        scratch += [
            pltpu.VMEM((t, _CH), jnp.int32),
            pltpu.VMEM((t, _CH), jnp.int32),
            pltpu.VMEM((t * _CH, _D), jnp.float32),
        ]
    scratch.append(pltpu.VMEM_SHARED((_R, _D), jnp.float32))
    if with_cnt:
        scratch += [
            pltpu.VMEM((_CH,), jnp.float32),
            pltpu.VMEM((_ZW,), jnp.float32),
            pltpu.VMEM_SHARED((_R + 16,), jnp.float32),
        ]
    scratch += [pltpu.SemaphoreType.DMA] * (8 if with_cnt else 6)

    @functools.partial(
        pl.kernel, mesh=_mesh(), out_type=out_type, scratch_types=scratch,
        compiler_params=pltpu.CompilerParams(use_tc_tiling_on_sc=False))
    def edge_pass(table, srcg, dstg, *rest):
        rest = list(rest)
        acc_out = rest.pop(0)
        cnt_out = rest.pop(0) if with_cnt else None
        idx_s = [rest[0], rest[3]]
        idx_d = [rest[1], rest[4]]
        rows = [rest[2], rest[5]]
        acc_sh = rest[6]
        if with_cnt:
            ones, zbuf, cnt_sh = rest[7], rest[8], rest[9]
            isem = rest[10:12]
            gsem = rest[12:14]
            ssem = rest[14:16]
            csem = rest[16:18]
        else:
            isem = rest[7:9]
            gsem = rest[9:11]
            ssem = rest[11:13]
        c = lax.axis_index("c")
        s = lax.axis_index("s")
        w = c * _NS + s

        @pl.loop(0, _ZC)
        def _(i):
            rows[0][i, :] = jnp.zeros((_D,), jnp.float32)

        for k in range(4):
            pltpu.sync_copy(rows[0].at[pl.ds(0, _ZC)],
                            acc_sh.at[pl.ds(s * _RPS + k * _ZC, _ZC)])
        if with_cnt:
            @pl.loop(0, _ZW // 16)
            def _(i):
                zbuf[pl.ds(i * 16, 16)] = jnp.zeros((16,), jnp.float32)

            for k in range(2):
                pltpu.sync_copy(zbuf,
                                cnt_sh.at[pl.ds(s * _RPS + k * _ZW, _ZW)])

            @pl.loop(0, _CH // 16)
            def _(i):
                ones[pl.ds(i * 16, 16)] = jnp.ones((16,), jnp.float32)

        plsc.subcore_barrier()

        base = jnp.where(c == 0, s * cps0, _NS * cps0 + s * cps1)
        pairs = jnp.where(c == 0, cps0 // (2 * t), cps1 // (2 * t))

        def load_idx(b, it):
            c0 = base + it * t
            pltpu.async_copy(srcg.at[pl.ds(c0, t)], idx_s[b], isem[b])
            pltpu.async_copy(dstg.at[pl.ds(c0, t)], idx_d[b], isem[b])

        def wait_idx(b):
            pltpu.make_async_copy(srcg.at[pl.ds(0, t)], idx_s[b],
                                  isem[b]).wait()
            pltpu.make_async_copy(dstg.at[pl.ds(0, t)], idx_d[b],
                                  isem[b]).wait()

        def run_gathers(b):
            hs = [pltpu.async_copy(table.at[idx_s[b].at[j]],
                                   rows[b].at[pl.ds(j * _CH, _CH)], gsem[b])
                  for j in range(t)]
            for h in hs:
                h.wait()

        def fire_scatters(b):
            for j in range(t):
                pltpu.async_copy(rows[b].at[pl.ds(j * _CH, _CH)],
                                 acc_sh.at[idx_d[b].at[j]], ssem[b],
                                 add=True)
                if with_cnt:
                    pltpu.async_copy(ones, cnt_sh.at[idx_d[b].at[j]],
                                     csem[b], add=True)

        def drain_scatters(b):
            for j in range(t):
                pltpu.make_async_copy(rows[b].at[pl.ds(j * _CH, _CH)],
                                      acc_sh.at[idx_d[b].at[j]],
                                      ssem[b]).wait()
                if with_cnt:
                    pltpu.make_async_copy(ones, cnt_sh.at[idx_d[b].at[j]],
                                          csem[b]).wait()

        load_idx(0, 0)

        @pl.loop(0, pairs)
        def _(p):
            it0 = p * 2
            wait_idx(0)
            run_gathers(0)

            @pl.when(p > 0)
            def _():
                drain_scatters(1)

            load_idx(1, it0 + 1)
            fire_scatters(0)
            wait_idx(1)
            run_gathers(1)
            drain_scatters(0)
            load_idx(0, it0 + 2)
            fire_scatters(1)

        drain_scatters(1)
        wait_idx(0)
        plsc.subcore_barrier()

        r0 = s * _RPS
        pltpu.sync_copy(acc_sh.at[pl.ds(r0, _RPS)],
                        acc_out.at[c, pl.ds(r0, _RPS)])
        if with_cnt:
            pltpu.sync_copy(cnt_sh.at[pl.ds(r0, _RPS)],
                            cnt_out.at[c, pl.ds(r0, _RPS)])

    return edge_pass


_edge_pass_cnt = _make_edge_pass(True, _T1, 928, 656)
_edge_pass = _make_edge_pass(False, _T2, 1044, 540)


@functools.partial(pl.kernel, mesh=_mesh(),
                   out_type=[jax.ShapeDtypeStruct((_GP, _D), jnp.float32),
                             jax.ShapeDtypeStruct((_GP,), jnp.float32)],
                   compiler_params=pltpu.CompilerParams(
                       use_tc_tiling_on_sc=False),
                   scratch_types=[
                       pltpu.VMEM((_CH, _D), jnp.float32),
                       pltpu.VMEM((1, _CH), jnp.int32),
                       pltpu.VMEM((_CH,), jnp.float32),
                       pltpu.VMEM((_GPS, _D), jnp.float32),
                       pltpu.VMEM((_GPS,), jnp.float32),
                       pltpu.VMEM_SHARED((_GP, _D), jnp.float32),
                       pltpu.VMEM_SHARED((_GP,), jnp.float32),
                   ])
def _pool_pass(hg, bg, psum_out, pcnt_out, hbuf, ibuf, ones, pbuf, cbuf,
               psum_sh, pcnt_sh):
    c = lax.axis_index("c")
    s = lax.axis_index("s")

    @pl.when(c == 0)
    def _():
        @pl.loop(0, _CH // 16)
        def _(i):
            ones[pl.ds(i * 16, 16)] = jnp.ones((16,), jnp.float32)

        @pl.loop(0, _GPS)
        def _(i):
            pbuf[i, :] = jnp.zeros((_D,), jnp.float32)

        @pl.loop(0, _GPS // 16)
        def _(i):
            cbuf[pl.ds(i * 16, 16)] = jnp.zeros((16,), jnp.float32)

        pltpu.sync_copy(pbuf, psum_sh.at[pl.ds(s * _GPS, _GPS)])
        pltpu.sync_copy(cbuf, pcnt_sh.at[pl.ds(s * _GPS, _GPS)])
        plsc.subcore_barrier()

        @pl.loop(0, _PPS)
        def _(it):
            ch = s * _PPS + it
            pltpu.sync_copy(hg.at[pl.ds(ch * _CH, _CH)], hbuf)
            pltpu.sync_copy(bg.at[ch], ibuf)
            pltpu.sync_copy(hbuf, psum_sh.at[ibuf.at[0]], add=True)
            pltpu.sync_copy(ones, pcnt_sh.at[ibuf.at[0]], add=True)

        plsc.subcore_barrier()

        pltpu.sync_copy(psum_sh.at[pl.ds(s * _GPS, _GPS)], pbuf)
        pltpu.sync_copy(pcnt_sh.at[pl.ds(s * _GPS, _GPS)], cbuf)
        pltpu.sync_copy(pbuf, psum_out.at[pl.ds(s * _GPS, _GPS)])
        pltpu.sync_copy(cbuf, pcnt_out.at[pl.ds(s * _GPS, _GPS)])


def _bdot(a, b):
    return jnp.dot(a.astype(jnp.bfloat16), b.astype(jnp.bfloat16),
                   preferred_element_type=jnp.float32)


def _qp_body(acc_ref, c0_ref, c1_ref, x_ref, w1lt_ref, b1_ref, w1rt_ref,
             h1_ref):
    cnt = jnp.maximum(c0_ref[:] + c1_ref[:], 1.0)
    mean = (acc_ref[0] + acc_ref[1]) / cnt
    h1 = _bdot(mean, w1lt_ref[:]) + b1_ref[:] + _bdot(x_ref[:], w1rt_ref[:])
    h1_ref[:] = jnp.maximum(h1, 0.0)


def _q2_body(acc_ref, c0_ref, c1_ref, h1_ref, w2lt_ref, b2_ref, w2rt_ref,
             h2_ref):
    cnt = jnp.maximum(c0_ref[:] + c1_ref[:], 1.0)
    mean = (acc_ref[0] + acc_ref[1]) / cnt
    h2_ref[:] = (_bdot(mean, w2lt_ref[:]) + b2_ref[:]
                 + _bdot(h1_ref[:], w2rt_ref[:]))


def _fin_body(psum_ref, pcnt_ref, wlint_ref, blin_ref, o_ref):
    cnt = jnp.maximum(pcnt_ref[:], 1.0)
    pooled = psum_ref[:] / cnt
    o_ref[:] = _bdot(pooled, wlint_ref[:]) + blin_ref[0, 0]


def _row_spec():
    return pl.BlockSpec((_BN, _D), lambda i: (i, 0))


def _vec_spec():
    return pl.BlockSpec((_BN, 1), lambda i: (i, 0))


def _w_spec():
    return pl.BlockSpec((_D, _D), lambda i: (0, 0))


def _b_spec():
    return pl.BlockSpec((1, _D), lambda i: (0, 0))


def _acc_spec():
    return pl.BlockSpec((_NC, _BN, _D), lambda i: (0, i, 0))


def kernel(matrix_encodings, edge_index, batch, W1l, b1l, W1r, W2l, b2l, W2r,
           Wlin, blin):
    x = matrix_encodings
    f32, i32 = jnp.float32, jnp.int32
    grid = (_N // _BN,)

    npad = _NCHUNKP * _CH - _E
    srcg = jnp.concatenate(
        [edge_index[0], jnp.zeros((npad,), i32)]).reshape(_NCHUNKP, _CH)
    dstg = jnp.concatenate(
        [edge_index[1], _N + (jnp.arange(npad, dtype=i32) % (_R - _N))]
    ).reshape(_NCHUNKP, _CH)

    acc1, cnt = _edge_pass_cnt(x, srcg, dstg)
    cnt0 = cnt[0].reshape(_R, 1)
    cnt1 = cnt[1].reshape(_R, 1)

    h1 = pl.pallas_call(
        _qp_body, grid=grid,
        in_specs=[_acc_spec(), _vec_spec(), _vec_spec(), _row_spec(),
                  _w_spec(), _b_spec(), _w_spec()],
        out_specs=_row_spec(),
        out_shape=jax.ShapeDtypeStruct((_N, _D), f32),
    )(acc1, cnt0, cnt1, x, W1l.T, b1l.reshape(1, _D), W1r.T)

    (acc2,) = _edge_pass(h1, srcg, dstg)

    h2 = pl.pallas_call(
        _q2_body, grid=grid,
        in_specs=[_acc_spec(), _vec_spec(), _vec_spec(), _row_spec(),
                  _w_spec(), _b_spec(), _w_spec()],
        out_specs=_row_spec(),
        out_shape=jax.ShapeDtypeStruct((_N, _D), f32),
    )(acc2, cnt0, cnt1, h1, W2l.T, b2l.reshape(1, _D), W2r.T)

    hg = jnp.concatenate([h2, jnp.zeros((_RP - _N, _D), f32)])
    bg = jnp.concatenate(
        [batch, jnp.full((_RP - _N,), _G, i32)]).reshape(_NPCH, 1, _CH)
    psum, pcnt = _pool_pass(hg, bg)
    out = pl.pallas_call(
        _fin_body, grid=(1,),
        in_specs=[pl.BlockSpec((_GP, _D), lambda i: (0, 0)),
                  pl.BlockSpec((_GP, 1), lambda i: (0, 0)),
                  pl.BlockSpec((_D, 1), lambda i: (0, 0)),
                  pl.BlockSpec((1, 1), lambda i: (0, 0))],
        out_specs=pl.BlockSpec((_GP, 1), lambda i: (0, 0)),
        out_shape=jax.ShapeDtypeStruct((_GP, 1), f32),
    )(psum, pcnt.reshape(_GP, 1), Wlin.T, blin.reshape(1, 1))
    return out[:_G]
